# baseline (speedup 1.0000x reference)
import jax
import jax.numpy as jnp
from jax.experimental import pallas as pl
from jax.experimental.pallas import tpu as pltpu


_PBLK = 256


def _add_body(ids_ref, hid_ref, emb_ref, gate_ref, out_ref):
    t = pl.program_id(1)
    g = jnp.tanh(gate_ref[0, 0])
    row = emb_ref[0, t, :]
    out_ref[...] = hid_ref[...] + g * row.reshape(1, 1, 1, emb_ref.shape[-1])


def kernel(hidden_state, aspect_ratio_ids, embedding_table, gate):
    B, T, P, H = hidden_state.shape
    V = embedding_table.shape[0]
    table = embedding_table.reshape(V, T, H)
    gate2d = gate.reshape(1, 1)
    ids = aspect_ratio_ids.astype(jnp.int32)
    np_ = pl.cdiv(P, _PBLK)

    grid_spec = pltpu.PrefetchScalarGridSpec(
        num_scalar_prefetch=1,
        grid=(B, T, np_),
        in_specs=[
            pl.BlockSpec((1, 1, _PBLK, H), lambda b, t, p, ids: (b, t, p, 0)),
            pl.BlockSpec((1, T, H), lambda b, t, p, ids: (ids[b], 0, 0)),
            pl.BlockSpec(memory_space=pltpu.SMEM),
        ],
        out_specs=pl.BlockSpec((1, 1, _PBLK, H), lambda b, t, p, ids: (b, t, p, 0)),
    )
    out = pl.pallas_call(
        _add_body,
        grid_spec=grid_spec,
        out_shape=jax.ShapeDtypeStruct((B, T, P, H), hidden_state.dtype),
        compiler_params=pltpu.CompilerParams(
            dimension_semantics=("parallel", "parallel", "parallel"),
        ),
    )(ids, hidden_state, table, gate2d)
    return out


# speedup vs baseline: 1.1350x; 1.1350x over previous
import jax
import jax.numpy as jnp
from jax.experimental import pallas as pl
from jax.experimental.pallas import tpu as pltpu

_K = 4


def _make_body(B, T, P, H):
    N = B * T

    def _body(ids_ref, gate_ref, hid_ref, table_ref, out_ref,
              in_bufs, out_bufs, in_sems, out_sems):
        g = jnp.tanh(gate_ref[0, 0])

        def in_copy(i):
            b, t = divmod(i, T)
            return pltpu.make_async_copy(
                hid_ref.at[b, t], in_bufs.at[i % _K], in_sems.at[i % _K])

        def out_copy(i):
            b, t = divmod(i, T)
            return pltpu.make_async_copy(
                out_bufs.at[i % _K], out_ref.at[b, t], out_sems.at[i % _K])

        for i in range(min(_K, N)):
            in_copy(i).start()
        for i in range(N):
            b, t = divmod(i, T)
            if i >= _K:
                out_copy(i - _K).wait()
            in_copy(i).wait()
            row = table_ref[ids_ref[b], t, :]
            out_bufs[i % _K] = in_bufs[i % _K] + g * row.reshape(1, H)
            out_copy(i).start()
            if i + _K < N:
                in_copy(i + _K).start()
        for i in range(max(N - _K, 0), N):
            out_copy(i).wait()

    return _body


def kernel(hidden_state, aspect_ratio_ids, embedding_table, gate):
    B, T, P, H = hidden_state.shape
    V = embedding_table.shape[0]
    table = embedding_table.reshape(V, T, H)
    gate2d = gate.reshape(1, 1)
    ids = aspect_ratio_ids.astype(jnp.int32)

    out = pl.pallas_call(
        _make_body(B, T, P, H),
        in_specs=[
            pl.BlockSpec(memory_space=pltpu.SMEM),
            pl.BlockSpec(memory_space=pltpu.SMEM),
            pl.BlockSpec(memory_space=pl.ANY),
            pl.BlockSpec(memory_space=pltpu.VMEM),
        ],
        out_specs=pl.BlockSpec(memory_space=pl.ANY),
        out_shape=jax.ShapeDtypeStruct((B, T, P, H), hidden_state.dtype),
        scratch_shapes=[
            pltpu.VMEM((_K, P, H), jnp.float32),
            pltpu.VMEM((_K, P, H), jnp.float32),
            pltpu.SemaphoreType.DMA((_K,)),
            pltpu.SemaphoreType.DMA((_K,)),
        ],
    )(ids, gate2d, hidden_state, table)
    return out
